# baseline (speedup 1.0000x reference)
import jax
import jax.numpy as jnp
from jax.experimental import pallas as pl
from jax.experimental.pallas import tpu as pltpu
from jax.experimental.pallas import tpu_sc as plsc

_N, _D, _H, _E = 4096, 768, 3072, 8
_MU, _G1, _G2, _B1, _B2 = 0.7, 1.0, 1.0, 0.9, 0.999
_BLK = 256
_PMAX = _N * 2 + _E * _BLK
_NBLK = _PMAX // _BLK
_W = 128
_DP = _D // 2


def _router_body(x_ref, wg_ref, bg_ref, i1_ref, i2_ref, p1_ref, p2_ref):
    logits = jax.lax.dot_general(
        x_ref[...].astype(jnp.bfloat16), wg_ref[...].astype(jnp.bfloat16),
        (((1,), (0,)), ((), ())),
        preferred_element_type=jnp.float32) + bg_ref[...]
    lane = jax.lax.broadcasted_iota(jnp.int32, logits.shape, 1)
    i1 = jnp.argmax(logits, axis=1)[:, None]
    m1 = jnp.max(logits, axis=1, keepdims=True)
    masked = jnp.where(lane == i1, -1e30, logits)
    i2 = jnp.argmax(masked, axis=1)[:, None]
    m2 = jnp.max(masked, axis=1, keepdims=True)
    t = jnp.exp(m2 - m1)
    p1 = 1.0 / (1.0 + t)
    i1_ref[...] = i1
    i2_ref[...] = i2
    p1_ref[...] = p1
    p2_ref[...] = 1.0 - p1


def _meta_body(i1_ref, i2_ref, pos1_ref, pos2_ref, gb_ref, nblk_ref):
    cb = 512
    f32, i32 = jnp.float32, jnp.int32
    tril = (jax.lax.broadcasted_iota(i32, (cb, cb), 0)
            > jax.lax.broadcasted_iota(i32, (cb, cb), 1)).astype(jnp.bfloat16)
    base = jnp.zeros((1, _E), f32)
    for iref, pref in ((i1_ref, pos1_ref), (i2_ref, pos2_ref)):
        for c in range(_N // cb):
            idx = iref[c * cb:(c + 1) * cb, :]
            oh = (jax.lax.broadcasted_iota(i32, (cb, _E), 1)
                  == idx).astype(jnp.bfloat16)
            prefix = jax.lax.dot_general(
                tril, oh, (((1,), (0,)), ((), ())),
                preferred_element_type=f32)
            ohf = oh.astype(f32)
            rank = jnp.sum((prefix + base) * ohf, axis=1, keepdims=True)
            pref[c * cb:(c + 1) * cb, :] = rank.astype(i32)
            base = base + jnp.sum(ohf, axis=0, keepdims=True)
    counts = base.astype(i32)
    pc = ((counts + (_BLK - 1)) // _BLK) * _BLK
    pcf = pc.astype(f32)
    triu8 = (jax.lax.broadcasted_iota(i32, (_E, _E), 0)
             < jax.lax.broadcasted_iota(i32, (_E, _E), 1)).astype(f32)
    off = jax.lax.dot_general(
        pcf, triu8, (((1,), (0,)), ((), ())),
        precision=jax.lax.Precision.HIGHEST)
    for iref, pref in ((i1_ref, pos1_ref), (i2_ref, pos2_ref)):
        ohf = (jax.lax.broadcasted_iota(i32, (_N, _E), 1)
               == iref[...]).astype(f32)
        pref[...] += jnp.sum(ohf * off, axis=1, keepdims=True).astype(i32)
    nblk_ref[...] = jnp.sum(pc // _BLK, axis=1, keepdims=True)
    blk_end = off + pcf
    eye8 = (jax.lax.broadcasted_iota(i32, (_E, _E), 0)
            == jax.lax.broadcasted_iota(i32, (_E, _E), 1)).astype(f32)
    blk_end_col = jax.lax.dot_general(
        eye8, blk_end, (((1,), (1,)), ((), ())),
        precision=jax.lax.Precision.HIGHEST)
    bi = (jax.lax.broadcasted_iota(i32, (_E, 128), 1) * _BLK).astype(f32)
    gbf = jnp.sum((bi >= blk_end_col).astype(f32), axis=0, keepdims=True)
    gb_ref[...] = jnp.minimum(gbf, _E - 1.0).astype(i32)


def _group_mm_body(gb_ref, nblk_ref, xs_ref, w1_ref, b1_ref, w2_ref, b2_ref,
                   ys_ref):
    del gb_ref
    b = pl.program_id(0)

    @pl.when(b < nblk_ref[0])
    def _():
        h = jnp.dot(xs_ref[...], w1_ref[0],
                    preferred_element_type=jnp.float32)
        h = jnp.maximum(h + b1_ref[0], 0.0).astype(jnp.bfloat16)
        eo = jnp.dot(h, w2_ref[0],
                     preferred_element_type=jnp.float32) + b2_ref[0]
        ys_ref[...] = eo.astype(jnp.bfloat16)


def _final_body(y0_ref, y1_ref, p1_ref, p2_ref, mom_ref, lnw_ref, lnb_ref,
                out_ref, p_ref, v_ref, mm_ref):
    mix = p1_ref[...] * y0_ref[...] + p2_ref[...] * y1_ref[...]
    mm = _MU * mom_ref[2] + _G2 * mix
    p = _B1 * mom_ref[0] + (1.0 - _B1) * mix
    v = _B2 * mom_ref[1] + (1.0 - _B2) * (mix * mix)
    y = -(_G1 / jnp.sqrt(v + 1e-8) * p)
    mean = jnp.mean(y, axis=1, keepdims=True)
    yc = y - mean
    var = jnp.mean(yc * yc, axis=1, keepdims=True)
    out_ref[...] = yc / jnp.sqrt(var + 1e-5) * lnw_ref[...] + lnb_ref[...]
    p_ref[...] = p
    v_ref[...] = v
    mm_ref[...] = mm


def _vmesh():
    return plsc.VectorSubcoreMesh(core_axis_name="c", subcore_axis_name="s")


def _pack(a):
    return jax.lax.bitcast_convert_type(
        a.reshape(a.shape[0], -1, 2), jnp.int32)


def _unpack(a):
    return jax.lax.bitcast_convert_type(
        a, jnp.bfloat16).reshape(a.shape[0], -1)


def _dispatch(xp, posflat):
    nsrc = _N // _W

    @pl.kernel(out_type=jax.ShapeDtypeStruct((_PMAX, _DP), xp.dtype),
               mesh=_vmesh())
    def k(x_hbm, i_hbm, o_hbm):
        def body(x_vmem, i_vmem):
            pltpu.sync_copy(x_vmem, o_hbm.at[i_vmem.at[0]])

        pltpu.emit_pipeline(
            body,
            grid=(2 * _N // _W,),
            in_specs=[
                pl.BlockSpec((_W, _DP), lambda i: (jax.lax.rem(i, nsrc), 0)),
                pl.BlockSpec((1, _W), lambda i: (0, i)),
            ],
            out_specs=[],
            core_axis_name=("c", "s"),
            dimension_semantics=(pltpu.PARALLEL,),
        )(x_hbm, i_hbm)

    return k(xp, posflat)


def _combine(ysp, posflat):
    @pl.kernel(out_type=jax.ShapeDtypeStruct((2 * _N, _DP), ysp.dtype),
               mesh=_vmesh())
    def k(ys_hbm, i_hbm, o_hbm):
        def body(i_vmem, o_vmem):
            pltpu.sync_copy(ys_hbm.at[i_vmem.at[0]], o_vmem)

        pltpu.emit_pipeline(
            body,
            grid=(2 * _N // _W,),
            in_specs=[pl.BlockSpec((1, _W), lambda i: (0, i))],
            out_specs=[pl.BlockSpec((_W, _DP), lambda i: (i, 0))],
            core_axis_name=("c", "s"),
            dimension_semantics=(pltpu.PARALLEL,),
        )(i_hbm, o_hbm)

    return k(ysp, posflat)


@jax.jit
def kernel(x, moment, W1, b1, W2, b2, Wg, bg, ln_w, ln_b):
    f32, i32 = jnp.float32, jnp.int32
    rb = 1024
    i1, i2, p1, p2 = pl.pallas_call(
        _router_body,
        grid=(_N // rb,),
        in_specs=[
            pl.BlockSpec((rb, _D), lambda i: (i, 0)),
            pl.BlockSpec((_D, _E), lambda i: (0, 0)),
            pl.BlockSpec((1, _E), lambda i: (0, 0)),
        ],
        out_specs=[pl.BlockSpec((rb, 1), lambda i: (i, 0))] * 4,
        out_shape=[jax.ShapeDtypeStruct((_N, 1), i32),
                   jax.ShapeDtypeStruct((_N, 1), i32),
                   jax.ShapeDtypeStruct((_N, 1), f32),
                   jax.ShapeDtypeStruct((_N, 1), f32)],
    )(x, Wg, bg.reshape(1, _E))

    pos1, pos2, gb, nblk = pl.pallas_call(
        _meta_body,
        grid=(1,),
        in_specs=[pl.BlockSpec((_N, 1), lambda i: (0, 0))] * 2,
        out_specs=[pl.BlockSpec((_N, 1), lambda i: (0, 0)),
                   pl.BlockSpec((_N, 1), lambda i: (0, 0)),
                   pl.BlockSpec((1, 128), lambda i: (0, 0)),
                   pl.BlockSpec((1, 1), lambda i: (0, 0))],
        out_shape=[jax.ShapeDtypeStruct((_N, 1), i32),
                   jax.ShapeDtypeStruct((_N, 1), i32),
                   jax.ShapeDtypeStruct((1, 128), i32),
                   jax.ShapeDtypeStruct((1, 1), i32)],
        compiler_params=pltpu.CompilerParams(
            dimension_semantics=("arbitrary",)),
    )(i1, i2)

    posflat = jnp.concatenate([pos1.reshape(1, _N), pos2.reshape(1, _N)],
                              axis=1)
    xs = _unpack(_dispatch(_pack(x.astype(jnp.bfloat16)), posflat))

    w1b = W1.astype(jnp.bfloat16)
    w2b = W2.astype(jnp.bfloat16)

    def _clamped(b, nb):
        return jnp.minimum(b, nb[0] - 1)

    ys = pl.pallas_call(
        _group_mm_body,
        grid_spec=pltpu.PrefetchScalarGridSpec(
            num_scalar_prefetch=2,
            grid=(_NBLK,),
            in_specs=[
                pl.BlockSpec((_BLK, _D),
                             lambda b, gb, nb: (_clamped(b, nb), 0)),
                pl.BlockSpec((1, _D, _H),
                             lambda b, gb, nb: (gb[_clamped(b, nb)], 0, 0)),
                pl.BlockSpec((1, 1, _H),
                             lambda b, gb, nb: (gb[_clamped(b, nb)], 0, 0)),
                pl.BlockSpec((1, _H, _D),
                             lambda b, gb, nb: (gb[_clamped(b, nb)], 0, 0)),
                pl.BlockSpec((1, 1, _D),
                             lambda b, gb, nb: (gb[_clamped(b, nb)], 0, 0)),
            ],
            out_specs=pl.BlockSpec((_BLK, _D),
                                   lambda b, gb, nb: (_clamped(b, nb), 0)),
        ),
        out_shape=jax.ShapeDtypeStruct((_PMAX, _D), jnp.bfloat16),
        compiler_params=pltpu.CompilerParams(
            dimension_semantics=("arbitrary",)),
    )(gb.reshape(128), nblk.reshape(1), xs, w1b,
      b1.reshape(_E, 1, _H), w2b, b2.reshape(_E, 1, _D))

    y01 = _unpack(_combine(_pack(ys), posflat))

    bn = 512
    shp = jax.ShapeDtypeStruct((_N, _D), f32)
    out, p, v, mm = pl.pallas_call(
        _final_body,
        grid=(_N // bn,),
        in_specs=[
            pl.BlockSpec((bn, _D), lambda n: (n, 0)),
            pl.BlockSpec((bn, _D), lambda n: (n + _N // bn, 0)),
            pl.BlockSpec((bn, 1), lambda n: (n, 0)),
            pl.BlockSpec((bn, 1), lambda n: (n, 0)),
            pl.BlockSpec((3, bn, _D), lambda n: (0, n, 0)),
            pl.BlockSpec((1, _D), lambda n: (0, 0)),
            pl.BlockSpec((1, _D), lambda n: (0, 0)),
        ],
        out_specs=[pl.BlockSpec((bn, _D), lambda n: (n, 0))] * 4,
        out_shape=[shp, shp, shp, shp],
    )(y01, y01, p1, p2, moment, ln_w.reshape(1, _D), ln_b.reshape(1, _D))
    return (out, p, v, mm)


# speedup vs baseline: 1.9155x; 1.9155x over previous
import jax
import jax.numpy as jnp
from jax.experimental import pallas as pl
from jax.experimental.pallas import tpu as pltpu
from jax.experimental.pallas import tpu_sc as plsc

_N, _D, _H, _E = 4096, 768, 3072, 8
_MU, _G1, _G2, _B1, _B2 = 0.7, 1.0, 1.0, 0.9, 0.999
_BLK = 256
_PMAX = _N * 2 + _E * _BLK
_NBLK = _PMAX // _BLK
_W = 128
_DH = _D // 2


def _router_body(x_ref, wg_ref, bg_ref, i1_ref, i2_ref, p1_ref, p2_ref):
    logits = jax.lax.dot_general(
        x_ref[...].astype(jnp.bfloat16), wg_ref[...].astype(jnp.bfloat16),
        (((1,), (0,)), ((), ())),
        preferred_element_type=jnp.float32) + bg_ref[...]
    lane = jax.lax.broadcasted_iota(jnp.int32, logits.shape, 1)
    i1 = jnp.argmax(logits, axis=1)[:, None]
    m1 = jnp.max(logits, axis=1, keepdims=True)
    masked = jnp.where(lane == i1, -1e30, logits)
    i2 = jnp.argmax(masked, axis=1)[:, None]
    m2 = jnp.max(masked, axis=1, keepdims=True)
    t = jnp.exp(m2 - m1)
    p1 = 1.0 / (1.0 + t)
    i1_ref[...] = i1
    i2_ref[...] = i2
    p1_ref[...] = p1
    p2_ref[...] = 1.0 - p1


def _meta_body(i1_ref, i2_ref, pos1_ref, pos2_ref, gb_ref, nblk_ref):
    cb = 512
    f32, i32 = jnp.float32, jnp.int32
    tril = (jax.lax.broadcasted_iota(i32, (cb, cb), 0)
            > jax.lax.broadcasted_iota(i32, (cb, cb), 1)).astype(jnp.bfloat16)
    base = jnp.zeros((1, _E), f32)
    for iref, pref in ((i1_ref, pos1_ref), (i2_ref, pos2_ref)):
        for c in range(_N // cb):
            idx = iref[c * cb:(c + 1) * cb, :]
            oh = (jax.lax.broadcasted_iota(i32, (cb, _E), 1)
                  == idx).astype(jnp.bfloat16)
            prefix = jax.lax.dot_general(
                tril, oh, (((1,), (0,)), ((), ())),
                preferred_element_type=f32)
            ohf = oh.astype(f32)
            rank = jnp.sum((prefix + base) * ohf, axis=1, keepdims=True)
            pref[c * cb:(c + 1) * cb, :] = rank.astype(i32)
            base = base + jnp.sum(ohf, axis=0, keepdims=True)
    counts = base.astype(i32)
    pc = ((counts + (_BLK - 1)) // _BLK) * _BLK
    pcf = pc.astype(f32)
    triu8 = (jax.lax.broadcasted_iota(i32, (_E, _E), 0)
             < jax.lax.broadcasted_iota(i32, (_E, _E), 1)).astype(f32)
    off = jax.lax.dot_general(
        pcf, triu8, (((1,), (0,)), ((), ())),
        precision=jax.lax.Precision.HIGHEST)
    for iref, pref in ((i1_ref, pos1_ref), (i2_ref, pos2_ref)):
        ohf = (jax.lax.broadcasted_iota(i32, (_N, _E), 1)
               == iref[...]).astype(f32)
        pref[...] += jnp.sum(ohf * off, axis=1, keepdims=True).astype(i32)
    nblk_ref[...] = jnp.sum(pc // _BLK, axis=1, keepdims=True)
    blk_end = off + pcf
    eye8 = (jax.lax.broadcasted_iota(i32, (_E, _E), 0)
            == jax.lax.broadcasted_iota(i32, (_E, _E), 1)).astype(f32)
    blk_end_col = jax.lax.dot_general(
        eye8, blk_end, (((1,), (1,)), ((), ())),
        precision=jax.lax.Precision.HIGHEST)
    bi = (jax.lax.broadcasted_iota(i32, (_E, 128), 1) * _BLK).astype(f32)
    gbf = jnp.sum((bi >= blk_end_col).astype(f32), axis=0, keepdims=True)
    gb_ref[...] = jnp.minimum(gbf, _E - 1.0).astype(i32)


def _group_mm_body(gb_ref, nblk_ref, xs_ref, w1_ref, b1_ref, w2_ref, b2_ref,
                   ys_ref):
    del gb_ref
    b = pl.program_id(0)

    @pl.when(b < nblk_ref[0])
    def _():
        h = jnp.dot(xs_ref[...].astype(jnp.bfloat16), w1_ref[0],
                    preferred_element_type=jnp.float32)
        h = jnp.maximum(h + b1_ref[0], 0.0).astype(jnp.bfloat16)
        ys_ref[...] = jnp.dot(h, w2_ref[0],
                              preferred_element_type=jnp.float32) + b2_ref[0]


def _final_body(y0_ref, y1_ref, p1_ref, p2_ref, mom_ref, lnw_ref, lnb_ref,
                out_ref, p_ref, v_ref, mm_ref):
    mix = p1_ref[...] * y0_ref[...] + p2_ref[...] * y1_ref[...]
    mm = _MU * mom_ref[2] + _G2 * mix
    p = _B1 * mom_ref[0] + (1.0 - _B1) * mix
    v = _B2 * mom_ref[1] + (1.0 - _B2) * (mix * mix)
    y = -(_G1 / jnp.sqrt(v + 1e-8) * p)
    mean = jnp.mean(y, axis=1, keepdims=True)
    yc = y - mean
    var = jnp.mean(yc * yc, axis=1, keepdims=True)
    out_ref[...] = yc / jnp.sqrt(var + 1e-5) * lnw_ref[...] + lnb_ref[...]
    p_ref[...] = p
    v_ref[...] = v
    mm_ref[...] = mm


def _vmesh():
    return plsc.VectorSubcoreMesh(core_axis_name="c", subcore_axis_name="s")


def _dispatch(xh, posh):
    nsrc = 2 * _N // _W

    @pl.kernel(out_type=jax.ShapeDtypeStruct((2 * _PMAX, _DH), xh.dtype),
               mesh=_vmesh())
    def k(x_hbm, i_hbm, o_hbm):
        def body(x_vmem, i_vmem):
            pltpu.sync_copy(x_vmem, o_hbm.at[i_vmem.at[0]])

        pltpu.emit_pipeline(
            body,
            grid=(4 * _N // _W,),
            in_specs=[
                pl.BlockSpec((_W, _DH), lambda i: (jax.lax.rem(i, nsrc), 0)),
                pl.BlockSpec((1, _W), lambda i: (0, i)),
            ],
            out_specs=[],
            core_axis_name=("c", "s"),
            dimension_semantics=(pltpu.PARALLEL,),
        )(x_hbm, i_hbm)

    return k(xh, posh)


def _combine(ys2, posh):
    @pl.kernel(out_type=jax.ShapeDtypeStruct((4 * _N, _DH), ys2.dtype),
               mesh=_vmesh())
    def k(ys_hbm, i_hbm, o_hbm):
        def body(i_vmem, o_vmem):
            pltpu.sync_copy(ys_hbm.at[i_vmem.at[0]], o_vmem)

        pltpu.emit_pipeline(
            body,
            grid=(4 * _N // _W,),
            in_specs=[pl.BlockSpec((1, _W), lambda i: (0, i))],
            out_specs=[pl.BlockSpec((_W, _DH), lambda i: (i, 0))],
            core_axis_name=("c", "s"),
            dimension_semantics=(pltpu.PARALLEL,),
        )(i_hbm, o_hbm)

    return k(ys2, posh)


@jax.jit
def kernel(x, moment, W1, b1, W2, b2, Wg, bg, ln_w, ln_b):
    f32, i32 = jnp.float32, jnp.int32
    rb = 1024
    i1, i2, p1, p2 = pl.pallas_call(
        _router_body,
        grid=(_N // rb,),
        in_specs=[
            pl.BlockSpec((rb, _D), lambda i: (i, 0)),
            pl.BlockSpec((_D, _E), lambda i: (0, 0)),
            pl.BlockSpec((1, _E), lambda i: (0, 0)),
        ],
        out_specs=[pl.BlockSpec((rb, 1), lambda i: (i, 0))] * 4,
        out_shape=[jax.ShapeDtypeStruct((_N, 1), i32),
                   jax.ShapeDtypeStruct((_N, 1), i32),
                   jax.ShapeDtypeStruct((_N, 1), f32),
                   jax.ShapeDtypeStruct((_N, 1), f32)],
    )(x, Wg, bg.reshape(1, _E))

    pos1, pos2, gb, nblk = pl.pallas_call(
        _meta_body,
        grid=(1,),
        in_specs=[pl.BlockSpec((_N, 1), lambda i: (0, 0))] * 2,
        out_specs=[pl.BlockSpec((_N, 1), lambda i: (0, 0)),
                   pl.BlockSpec((_N, 1), lambda i: (0, 0)),
                   pl.BlockSpec((1, 128), lambda i: (0, 0)),
                   pl.BlockSpec((1, 1), lambda i: (0, 0))],
        out_shape=[jax.ShapeDtypeStruct((_N, 1), i32),
                   jax.ShapeDtypeStruct((_N, 1), i32),
                   jax.ShapeDtypeStruct((1, 128), i32),
                   jax.ShapeDtypeStruct((1, 1), i32)],
        compiler_params=pltpu.CompilerParams(
            dimension_semantics=("arbitrary",)),
    )(i1, i2)

    ph1 = jnp.concatenate([2 * pos1, 2 * pos1 + 1], axis=1).reshape(1, 2 * _N)
    ph2 = jnp.concatenate([2 * pos2, 2 * pos2 + 1], axis=1).reshape(1, 2 * _N)
    posh = jnp.concatenate([ph1, ph2], axis=1)
    xs = _dispatch(x.reshape(2 * _N, _DH), posh).reshape(_PMAX, _D)

    w1b = W1.astype(jnp.bfloat16)
    w2b = W2.astype(jnp.bfloat16)

    def _clamped(b, nb):
        return jnp.minimum(b, nb[0] - 1)

    ys = pl.pallas_call(
        _group_mm_body,
        grid_spec=pltpu.PrefetchScalarGridSpec(
            num_scalar_prefetch=2,
            grid=(_NBLK,),
            in_specs=[
                pl.BlockSpec((_BLK, _D),
                             lambda b, gb, nb: (_clamped(b, nb), 0)),
                pl.BlockSpec((1, _D, _H),
                             lambda b, gb, nb: (gb[_clamped(b, nb)], 0, 0)),
                pl.BlockSpec((1, 1, _H),
                             lambda b, gb, nb: (gb[_clamped(b, nb)], 0, 0)),
                pl.BlockSpec((1, _H, _D),
                             lambda b, gb, nb: (gb[_clamped(b, nb)], 0, 0)),
                pl.BlockSpec((1, 1, _D),
                             lambda b, gb, nb: (gb[_clamped(b, nb)], 0, 0)),
            ],
            out_specs=pl.BlockSpec((_BLK, _D),
                                   lambda b, gb, nb: (_clamped(b, nb), 0)),
        ),
        out_shape=jax.ShapeDtypeStruct((_PMAX, _D), f32),
        compiler_params=pltpu.CompilerParams(
            dimension_semantics=("arbitrary",)),
    )(gb.reshape(128), nblk.reshape(1), xs, w1b,
      b1.reshape(_E, 1, _H), w2b, b2.reshape(_E, 1, _D))

    y01 = _combine(ys.reshape(2 * _PMAX, _DH), posh).reshape(2 * _N, _D)

    bn = 512
    shp = jax.ShapeDtypeStruct((_N, _D), f32)
    out, p, v, mm = pl.pallas_call(
        _final_body,
        grid=(_N // bn,),
        in_specs=[
            pl.BlockSpec((bn, _D), lambda n: (n, 0)),
            pl.BlockSpec((bn, _D), lambda n: (n + _N // bn, 0)),
            pl.BlockSpec((bn, 1), lambda n: (n, 0)),
            pl.BlockSpec((bn, 1), lambda n: (n, 0)),
            pl.BlockSpec((3, bn, _D), lambda n: (0, n, 0)),
            pl.BlockSpec((1, _D), lambda n: (0, 0)),
            pl.BlockSpec((1, _D), lambda n: (0, 0)),
        ],
        out_specs=[pl.BlockSpec((bn, _D), lambda n: (n, 0))] * 4,
        out_shape=[shp, shp, shp, shp],
    )(y01, y01, p1, p2, moment, ln_w.reshape(1, _D), ln_b.reshape(1, _D))
    return (out, p, v, mm)
